# baseline (speedup 1.0000x reference)
import jax
import jax.numpy as jnp
import numpy as np
from jax import lax
from jax.experimental import pallas as pl
from jax.experimental.pallas import tpu as pltpu
from jax.experimental.pallas import tpu_sc as plsc

N, E, D = 10000, 320000, 128
NPAD = 10240
EPAD = 327680
B = 64
SB = 20
NBUF = 5
NBLK_C = EPAD // 16 // B
NSB = NBLK_C // SB
BA = 128
NBLK_A = EPAD // 32 // BA
CHUNK_C = NBLK_C * B
CHUNK_A = EPAD // 32
ROWS_T = NPAD // 16


def _deg_body(tidx_hbm, ew_hbm, degp_hbm, tidx_v, ew_v, zbuf_v, deg_s):
    c = lax.axis_index("c")
    s = lax.axis_index("s")
    w = c * 16 + s
    pltpu.sync_copy(tidx_hbm.at[w], tidx_v)
    pltpu.sync_copy(ew_hbm.at[w], ew_v)

    @pl.loop(0, 8)
    def _(k):
        zbuf_v[pl.ds(k * 16, 16)] = jnp.zeros((16,), jnp.float32)

    @pl.loop(0, ROWS_T // BA)
    def _(r):
        pltpu.sync_copy(zbuf_v, deg_s.at[pl.ds(s * ROWS_T + r * BA, BA)])

    plsc.subcore_barrier()

    @pl.loop(0, NBLK_A)
    def _(j):
        pltpu.sync_copy(ew_v.at[j], deg_s.at[tidx_v.at[j]], add=True)

    plsc.subcore_barrier()
    pltpu.sync_copy(deg_s.at[pl.ds(s * ROWS_T, ROWS_T)],
                    degp_hbm.at[c, pl.ds(s * ROWS_T, ROWS_T)])


_deg_call = pl.kernel(
    _deg_body,
    out_type=jax.ShapeDtypeStruct((2, NPAD), jnp.float32),
    mesh=plsc.VectorSubcoreMesh(core_axis_name="c", subcore_axis_name="s"),
    compiler_params=pltpu.CompilerParams(needs_layout_passes=False),
    scratch_types=[
        pltpu.VMEM((NBLK_A, BA), jnp.int32),
        pltpu.VMEM((NBLK_A, BA), jnp.float32),
        pltpu.VMEM((BA,), jnp.float32),
        pltpu.VMEM_SHARED((NPAD,), jnp.float32),
    ],
)


def _coef_body(sidx_hbm, ew_hbm, dinv_hbm, c1_hbm, sidx_v, ew_v, dinv_v, c1_v):
    c = lax.axis_index("c")
    s = lax.axis_index("s")
    w = c * 16 + s
    pltpu.sync_copy(sidx_hbm.at[w], sidx_v)
    pltpu.sync_copy(ew_hbm.at[w], ew_v)
    pltpu.sync_copy(dinv_hbm, dinv_v)

    @pl.loop(0, CHUNK_A // 16)
    def _(i):
        sl = pl.ds(i * 16, 16)
        dv = plsc.load_gather(dinv_v, [sidx_v[sl]])
        c1_v[sl] = dv * ew_v[sl]

    pltpu.sync_copy(c1_v, c1_hbm.at[w])


_coef_call = pl.kernel(
    _coef_body,
    out_type=jax.ShapeDtypeStruct((32, CHUNK_A), jnp.float32),
    mesh=plsc.VectorSubcoreMesh(core_axis_name="c", subcore_axis_name="s"),
    compiler_params=pltpu.CompilerParams(needs_layout_passes=False),
    scratch_types=[
        pltpu.VMEM((CHUNK_A,), jnp.int32),
        pltpu.VMEM((CHUNK_A,), jnp.float32),
        pltpu.VMEM((NPAD,), jnp.float32),
        pltpu.VMEM((CHUNK_A,), jnp.float32),
    ],
)


_HI = np.uint32(0xFFFF0000)


def _spmm_body(sidx_hbm, tidx_hbm, coef_hbm, x_hbm, z_hbm,
               sidx_v, tidx_v, coef_v, rows_v, sc_v, acc_s, gsems, scsems):
    c = lax.axis_index("c")
    s = lax.axis_index("s")

    z0 = sc_v[0]

    @pl.loop(0, B)
    def _(i):
        for k in range(8):
            z0[i, pl.ds(k * 16, 16)] = jnp.zeros((16,), jnp.float32)

    @pl.loop(0, ROWS_T // B)
    def _(r):
        pltpu.sync_copy(z0, acc_s.at[pl.ds(s * ROWS_T + r * B, B)])

    plsc.subcore_barrier()

    def gather_src(j):
        return x_hbm.at[sidx_v.at[pl.ds(j * B, B)]]

    def scatter_dst(tp, j):
        return acc_s.at[tidx_v.at[tp, j]]

    def drain_scatter(u, tp, j):
        pltpu.make_async_copy(sc_v[u], scatter_dst(tp, j), scsems[u]).wait()

    @pl.loop(0, NSB // 2)
    def _(th):
        for tp in range(2):
            t = th * 2 + tp
            pltpu.sync_copy(sidx_hbm.at[s, pl.ds(t * SB * B, SB * B)], sidx_v)
            pltpu.sync_copy(tidx_hbm.at[s, pl.ds(t * SB, SB)],
                            tidx_v.at[tp])
            pltpu.sync_copy(coef_hbm.at[c, s, pl.ds(t * SB * B, SB * B)],
                            coef_v)

            for g in range(NBUF - 1):
                pltpu.async_copy(gather_src(g), rows_v[g], gsems[g])

            @pl.loop(0, SB // NBUF)
            def _(q):
                for b in range(NBUF):
                    j = q * NBUF + b
                    u = b
                    up = (b + NBUF - 1) % NBUF
                    v = b % 2

                    @pl.when(j + NBUF - 1 < SB)
                    def _():
                        pltpu.async_copy(gather_src(j + NBUF - 1), rows_v[up],
                                         gsems[up])

                    @pl.when((t > 0) | (j > 1))
                    def _():
                        drain_scatter(v, tp, j)

                    pltpu.make_async_copy(gather_src(j), rows_v[u],
                                          gsems[u]).wait()

                    rb, sb = rows_v[u], sc_v[v]

                    @pl.loop(0, B)
                    def _(i):
                        cf = plsc.load_gather(
                            coef_v, [jnp.full((16,), j * B + i, jnp.int32)])
                        for k in range(4):
                            u32 = plsc.bitcast(rb[i, pl.ds(16 * k, 16)],
                                               jnp.uint32)
                            lo = plsc.bitcast(u32 << 16, jnp.float32)
                            hi = plsc.bitcast(u32 & _HI, jnp.float32)
                            sb[i, pl.ds(32 * k, 16)] = lo * cf
                            sb[i, pl.ds(32 * k + 16, 16)] = hi * cf

                    pltpu.async_copy(sb, scatter_dst(tp, j), scsems[v],
                                     add=True)

    for v in range(2):
        drain_scatter(v, (NSB - 1) % 2, 0)
    plsc.subcore_barrier()
    pltpu.sync_copy(acc_s.at[pl.ds(s * ROWS_T, ROWS_T)],
                    z_hbm.at[c, pl.ds(s * ROWS_T, ROWS_T)])


def _spmm_wrap(sidx_hbm, tidx_hbm, coef_hbm, x_hbm, z_hbm,
               sidx_v, tidx_v, coef_v, r0, r1, r2, r3, r4, sc0, sc1, acc_s,
               g0, g1, g2, g3, g4, s0, s1):
    _spmm_body(sidx_hbm, tidx_hbm, coef_hbm, x_hbm, z_hbm,
               sidx_v, tidx_v, coef_v, (r0, r1, r2, r3, r4), (sc0, sc1),
               acc_s, (g0, g1, g2, g3, g4), (s0, s1))


_spmm_call = pl.kernel(
    _spmm_wrap,
    out_type=jax.ShapeDtypeStruct((2, NPAD, D), jnp.float32),
    mesh=plsc.VectorSubcoreMesh(core_axis_name="c", subcore_axis_name="s"),
    compiler_params=pltpu.CompilerParams(needs_layout_passes=False,
                                         use_tc_tiling_on_sc=False),
    scratch_types=(
        [pltpu.VMEM((SB * B,), jnp.int32),
         pltpu.VMEM((2, SB, B), jnp.int32),
         pltpu.VMEM((SB * B,), jnp.float32)]
        + [pltpu.VMEM((B, D // 2), jnp.int32)] * NBUF
        + [pltpu.VMEM((B, D), jnp.float32)] * 2
        + [pltpu.VMEM_SHARED((NPAD, D), jnp.float32)]
        + [pltpu.SemaphoreType.DMA] * (NBUF + 2)
    ),
)

_FPERM = tuple(32 * k + (2 * (i - 16) + 1 if i >= 16 else 2 * i)
               for k in range(4) for i in range(32))


def _mid_body(degp, wgcn, wihf, bihf, bhhf, wihb, bihb, bhhb, wcom, wfuse,
              dinv_o, dinv2_o, rdeg_o, w1_o, w2_o):
    deg = degp[0, :] + degp[1, :] + 1.0
    dinv = jnp.where(deg > 0, lax.rsqrt(deg), 0.0)
    dinv_o[...] = dinv
    dinv2_o[...] = dinv * dinv
    rdeg_o[...] = 1.0 / jnp.maximum(deg, 1.0)

    wg = wgcn[...]

    def lstm(wih, bih, bhh):
        g = lax.dot_general(wg, wih[...], (((1,), (1,)), ((), ())),
                            preferred_element_type=jnp.float32)
        g = g + bih[...] + bhh[...]
        i, f, gg, o = g[:, 0:D], g[:, D:2 * D], g[:, 2 * D:3 * D], g[:, 3 * D:4 * D]
        cst = jax.nn.sigmoid(i) * jnp.tanh(gg)
        return jax.nn.sigmoid(o) * jnp.tanh(cst)

    w_ev = jnp.maximum(lstm(wihf, bihf, bhhf), lstm(wihb, bihb, bhhb))
    w1_o[...] = lax.dot_general(w_ev, wfuse[...], (((1,), (1,)), ((), ())),
                                preferred_element_type=jnp.float32)
    w2_o[...] = lax.dot_general(wcom[...], wfuse[...], (((1,), (1,)), ((), ())),
                                preferred_element_type=jnp.float32)


def _mid_call(degp, wgcn, wihf, bihf, bhhf, wihb, bihb, bhhb, wcom, wfuse):
    return pl.pallas_call(
        _mid_body,
        out_shape=[
            jax.ShapeDtypeStruct((NPAD,), jnp.float32),
            jax.ShapeDtypeStruct((NPAD,), jnp.float32),
            jax.ShapeDtypeStruct((NPAD,), jnp.float32),
            jax.ShapeDtypeStruct((D, D), jnp.float32),
            jax.ShapeDtypeStruct((D, D), jnp.float32),
        ],
    )(degp, wgcn, wihf, bihf, bhhf, wihb, bihb, bhhb, wcom, wfuse)


_RB = 400


def _fuse_body(x, xne, z1, z2, dv, dv2, rd, wf, w1, w2, bf, o):
    xb = x[...]
    g2 = z1[...] * dv[...] + xb * dv2[...]
    g3 = (z2[...] + xb) * rd[...]
    acc = lax.dot_general(xne[...], wf[...], (((1,), (1,)), ((), ())),
                          preferred_element_type=jnp.float32)
    acc = acc + lax.dot_general(g2, w1[...], (((1,), (0,)), ((), ())),
                                preferred_element_type=jnp.float32)
    acc = acc + lax.dot_general(g3, w2[...], (((1,), (0,)), ((), ())),
                                preferred_element_type=jnp.float32)
    o[...] = acc + bf[...]


def _fuse_call(x, xne, z1, z2, dinv, dinv2, rdeg, wf, w1, w2, bf):
    col = pl.BlockSpec((_RB, 1), lambda i: (i, 0))
    mat = pl.BlockSpec((_RB, D), lambda i: (i, 0))
    w = pl.BlockSpec((D, D), lambda i: (0, 0))
    return pl.pallas_call(
        _fuse_body,
        grid=(N // _RB,),
        in_specs=[mat, mat, mat, mat, col, col, col, w, w, w,
                  pl.BlockSpec((1, D), lambda i: (0, 0))],
        out_specs=mat,
        out_shape=jax.ShapeDtypeStruct((N, D), jnp.float32),
    )(x, xne, z1, z2, dinv, dinv2, rdeg, wf, w1, w2, bf)


def kernel(X, edge_index, edge_weight, X_node_emb, W_gcn, W_ih_f, W_hh_f,
           b_ih_f, b_hh_f, W_ih_b, W_hh_b, b_ih_b, b_hh_b, W_com, W_fuse,
           b_fuse):
    pad = EPAD - E
    sidx = jnp.concatenate([edge_index[0], jnp.zeros((pad,), jnp.int32)])
    tidx = jnp.concatenate([edge_index[1], jnp.zeros((pad,), jnp.int32)])
    ewp = jnp.concatenate([edge_weight, jnp.zeros((pad,), jnp.float32)])

    degp = _deg_call(tidx.reshape(32, NBLK_A, BA), ewp.reshape(32, NBLK_A, BA))

    dinv, dinv2, rdeg, w1, w2 = _mid_call(
        degp, W_gcn, W_ih_f, b_ih_f, b_hh_f, W_ih_b, b_ih_b, b_hh_b,
        W_com, W_fuse)

    c1 = _coef_call(sidx.reshape(32, CHUNK_A), ewp.reshape(32, CHUNK_A),
                    dinv).reshape(EPAD)

    coefs = jnp.stack([c1, ewp]).reshape(2, 16, CHUNK_C)

    z = _spmm_call(sidx.reshape(16, CHUNK_C), tidx.reshape(16, NBLK_C, B),
                   coefs,
                   lax.bitcast_convert_type(
                       X.astype(jnp.bfloat16).reshape(N, D // 2, 2),
                       jnp.int32))

    fp = np.array(_FPERM, np.int32)
    return _fuse_call(X[:, fp], X_node_emb, z[0], z[1],
                      dinv.reshape(NPAD, 1), dinv2.reshape(NPAD, 1),
                      rdeg.reshape(NPAD, 1), W_fuse, w1[fp, :], w2[fp, :],
                      b_fuse.reshape(1, D))


# speedup vs baseline: 1.0288x; 1.0288x over previous
import jax
import jax.numpy as jnp
import numpy as np
from jax import lax
from jax.experimental import pallas as pl
from jax.experimental.pallas import tpu as pltpu
from jax.experimental.pallas import tpu_sc as plsc

N, E, D = 10000, 320000, 128
NPAD = 10240
EPAD = 327680
B = 64
SB = 40
NBUF = 4
NBLK_C = EPAD // 16 // B
NSB = NBLK_C // SB
BA = 128
NBLK_A = EPAD // 32 // BA
CHUNK_C = NBLK_C * B
CHUNK_A = EPAD // 32
ROWS_T = NPAD // 16


def _deg_body(tidx_hbm, ew_hbm, degp_hbm, tidx_v, ew_v, zbuf_v, deg_s):
    c = lax.axis_index("c")
    s = lax.axis_index("s")
    w = c * 16 + s
    pltpu.sync_copy(tidx_hbm.at[w], tidx_v)
    pltpu.sync_copy(ew_hbm.at[w], ew_v)

    @pl.loop(0, 8)
    def _(k):
        zbuf_v[pl.ds(k * 16, 16)] = jnp.zeros((16,), jnp.float32)

    @pl.loop(0, ROWS_T // BA)
    def _(r):
        pltpu.sync_copy(zbuf_v, deg_s.at[pl.ds(s * ROWS_T + r * BA, BA)])

    plsc.subcore_barrier()

    @pl.loop(0, NBLK_A)
    def _(j):
        pltpu.sync_copy(ew_v.at[j], deg_s.at[tidx_v.at[j]], add=True)

    plsc.subcore_barrier()
    pltpu.sync_copy(deg_s.at[pl.ds(s * ROWS_T, ROWS_T)],
                    degp_hbm.at[c, pl.ds(s * ROWS_T, ROWS_T)])


_deg_call = pl.kernel(
    _deg_body,
    out_type=jax.ShapeDtypeStruct((2, NPAD), jnp.float32),
    mesh=plsc.VectorSubcoreMesh(core_axis_name="c", subcore_axis_name="s"),
    compiler_params=pltpu.CompilerParams(needs_layout_passes=False),
    scratch_types=[
        pltpu.VMEM((NBLK_A, BA), jnp.int32),
        pltpu.VMEM((NBLK_A, BA), jnp.float32),
        pltpu.VMEM((BA,), jnp.float32),
        pltpu.VMEM_SHARED((NPAD,), jnp.float32),
    ],
)


def _coef_body(sidx_hbm, ew_hbm, dinv_hbm, c1_hbm, sidx_v, ew_v, dinv_v, c1_v):
    c = lax.axis_index("c")
    s = lax.axis_index("s")
    w = c * 16 + s
    pltpu.sync_copy(sidx_hbm.at[w], sidx_v)
    pltpu.sync_copy(ew_hbm.at[w], ew_v)
    pltpu.sync_copy(dinv_hbm, dinv_v)

    @pl.loop(0, CHUNK_A // 16)
    def _(i):
        sl = pl.ds(i * 16, 16)
        dv = plsc.load_gather(dinv_v, [sidx_v[sl]])
        c1_v[sl] = dv * ew_v[sl]

    pltpu.sync_copy(c1_v, c1_hbm.at[w])


_coef_call = pl.kernel(
    _coef_body,
    out_type=jax.ShapeDtypeStruct((32, CHUNK_A), jnp.float32),
    mesh=plsc.VectorSubcoreMesh(core_axis_name="c", subcore_axis_name="s"),
    compiler_params=pltpu.CompilerParams(needs_layout_passes=False),
    scratch_types=[
        pltpu.VMEM((CHUNK_A,), jnp.int32),
        pltpu.VMEM((CHUNK_A,), jnp.float32),
        pltpu.VMEM((NPAD,), jnp.float32),
        pltpu.VMEM((CHUNK_A,), jnp.float32),
    ],
)


_HI = np.uint32(0xFFFF0000)


def _spmm_body(sidx_hbm, tidx_hbm, coef_hbm, x_hbm, z_hbm,
               sidx_v, tidx_v, coef_v, rows_v, sc_v, acc_s, gsems, scsems):
    c = lax.axis_index("c")
    s = lax.axis_index("s")

    z0 = sc_v[0]

    @pl.loop(0, B)
    def _(i):
        for k in range(8):
            z0[i, pl.ds(k * 16, 16)] = jnp.zeros((16,), jnp.float32)

    @pl.loop(0, ROWS_T // B)
    def _(r):
        pltpu.sync_copy(z0, acc_s.at[pl.ds(s * ROWS_T + r * B, B)])

    plsc.subcore_barrier()

    def gather_src(j):
        return x_hbm.at[sidx_v.at[pl.ds(j * B, B)]]

    def scatter_dst(tp, j):
        return acc_s.at[tidx_v.at[tp, j]]

    def drain_scatter(u, tp, j):
        pltpu.make_async_copy(sc_v[u], scatter_dst(tp, j), scsems[u]).wait()

    @pl.loop(0, NSB // 2)
    def _(th):
        for tp in range(2):
            t = th * 2 + tp
            pltpu.sync_copy(sidx_hbm.at[s, pl.ds(t * SB * B, SB * B)], sidx_v)
            pltpu.sync_copy(tidx_hbm.at[s, pl.ds(t * SB, SB)],
                            tidx_v.at[tp])
            pltpu.sync_copy(coef_hbm.at[c, s, pl.ds(t * SB * B, SB * B)],
                            coef_v)

            for g in range(NBUF - 1):
                pltpu.async_copy(gather_src(g), rows_v[g], gsems[g])

            @pl.loop(0, SB // NBUF)
            def _(q):
                for b in range(NBUF):
                    j = q * NBUF + b
                    u = b
                    up = (b + NBUF - 1) % NBUF
                    v = b % 2

                    @pl.when(j + NBUF - 1 < SB)
                    def _():
                        pltpu.async_copy(gather_src(j + NBUF - 1), rows_v[up],
                                         gsems[up])

                    @pl.when((t > 0) | (j > 1))
                    def _():
                        drain_scatter(v, tp, j)

                    pltpu.make_async_copy(gather_src(j), rows_v[u],
                                          gsems[u]).wait()

                    rb, sb = rows_v[u], sc_v[v]

                    @pl.loop(0, B)
                    def _(i):
                        cf = plsc.load_gather(
                            coef_v, [jnp.full((16,), j * B + i, jnp.int32)])
                        for k in range(4):
                            u32 = plsc.bitcast(rb[i, pl.ds(16 * k, 16)],
                                               jnp.uint32)
                            lo = plsc.bitcast(u32 << 16, jnp.float32)
                            hi = plsc.bitcast(u32 & _HI, jnp.float32)
                            sb[i, pl.ds(32 * k, 16)] = lo * cf
                            sb[i, pl.ds(32 * k + 16, 16)] = hi * cf

                    pltpu.async_copy(sb, scatter_dst(tp, j), scsems[v],
                                     add=True)

    for v in range(2):
        drain_scatter(v, (NSB - 1) % 2, 0)
    plsc.subcore_barrier()
    pltpu.sync_copy(acc_s.at[pl.ds(s * ROWS_T, ROWS_T)],
                    z_hbm.at[c, pl.ds(s * ROWS_T, ROWS_T)])


def _spmm_wrap(sidx_hbm, tidx_hbm, coef_hbm, x_hbm, z_hbm,
               sidx_v, tidx_v, coef_v, r0, r1, r2, r3, sc0, sc1, acc_s,
               g0, g1, g2, g3, s0, s1):
    _spmm_body(sidx_hbm, tidx_hbm, coef_hbm, x_hbm, z_hbm,
               sidx_v, tidx_v, coef_v, (r0, r1, r2, r3), (sc0, sc1), acc_s,
               (g0, g1, g2, g3), (s0, s1))


_spmm_call = pl.kernel(
    _spmm_wrap,
    out_type=jax.ShapeDtypeStruct((2, NPAD, D), jnp.float32),
    mesh=plsc.VectorSubcoreMesh(core_axis_name="c", subcore_axis_name="s"),
    compiler_params=pltpu.CompilerParams(needs_layout_passes=False,
                                         use_tc_tiling_on_sc=False),
    scratch_types=(
        [pltpu.VMEM((SB * B,), jnp.int32),
         pltpu.VMEM((2, SB, B), jnp.int32),
         pltpu.VMEM((SB * B,), jnp.float32)]
        + [pltpu.VMEM((B, D // 2), jnp.int32)] * NBUF
        + [pltpu.VMEM((B, D), jnp.float32)] * 2
        + [pltpu.VMEM_SHARED((NPAD, D), jnp.float32)]
        + [pltpu.SemaphoreType.DMA] * (NBUF + 2)
    ),
)

_FPERM = tuple(32 * k + (2 * (i - 16) + 1 if i >= 16 else 2 * i)
               for k in range(4) for i in range(32))


def _mid_body(degp, wgcn, wihf, bihf, bhhf, wihb, bihb, bhhb, wcom, wfuse,
              dinv_o, dinv2_o, rdeg_o, w1_o, w2_o):
    deg = degp[0, :] + degp[1, :] + 1.0
    dinv = jnp.where(deg > 0, lax.rsqrt(deg), 0.0)
    dinv_o[...] = dinv
    dinv2_o[...] = dinv * dinv
    rdeg_o[...] = 1.0 / jnp.maximum(deg, 1.0)

    wg = wgcn[...]

    def lstm(wih, bih, bhh):
        g = lax.dot_general(wg, wih[...], (((1,), (1,)), ((), ())),
                            preferred_element_type=jnp.float32)
        g = g + bih[...] + bhh[...]
        i, f, gg, o = g[:, 0:D], g[:, D:2 * D], g[:, 2 * D:3 * D], g[:, 3 * D:4 * D]
        cst = jax.nn.sigmoid(i) * jnp.tanh(gg)
        return jax.nn.sigmoid(o) * jnp.tanh(cst)

    w_ev = jnp.maximum(lstm(wihf, bihf, bhhf), lstm(wihb, bihb, bhhb))
    w1_o[...] = lax.dot_general(w_ev, wfuse[...], (((1,), (1,)), ((), ())),
                                preferred_element_type=jnp.float32)
    w2_o[...] = lax.dot_general(wcom[...], wfuse[...], (((1,), (1,)), ((), ())),
                                preferred_element_type=jnp.float32)


def _mid_call(degp, wgcn, wihf, bihf, bhhf, wihb, bihb, bhhb, wcom, wfuse):
    return pl.pallas_call(
        _mid_body,
        out_shape=[
            jax.ShapeDtypeStruct((NPAD,), jnp.float32),
            jax.ShapeDtypeStruct((NPAD,), jnp.float32),
            jax.ShapeDtypeStruct((NPAD,), jnp.float32),
            jax.ShapeDtypeStruct((D, D), jnp.float32),
            jax.ShapeDtypeStruct((D, D), jnp.float32),
        ],
    )(degp, wgcn, wihf, bihf, bhhf, wihb, bihb, bhhb, wcom, wfuse)


_RB = 400


def _fuse_body(x, xne, z1, z2, dv, dv2, rd, wf, w1, w2, bf, o):
    xb = x[...]
    g2 = z1[...] * dv[...] + xb * dv2[...]
    g3 = (z2[...] + xb) * rd[...]
    acc = lax.dot_general(xne[...], wf[...], (((1,), (1,)), ((), ())),
                          preferred_element_type=jnp.float32)
    acc = acc + lax.dot_general(g2, w1[...], (((1,), (0,)), ((), ())),
                                preferred_element_type=jnp.float32)
    acc = acc + lax.dot_general(g3, w2[...], (((1,), (0,)), ((), ())),
                                preferred_element_type=jnp.float32)
    o[...] = acc + bf[...]


def _fuse_call(x, xne, z1, z2, dinv, dinv2, rdeg, wf, w1, w2, bf):
    col = pl.BlockSpec((_RB, 1), lambda i: (i, 0))
    mat = pl.BlockSpec((_RB, D), lambda i: (i, 0))
    w = pl.BlockSpec((D, D), lambda i: (0, 0))
    return pl.pallas_call(
        _fuse_body,
        grid=(N // _RB,),
        in_specs=[mat, mat, mat, mat, col, col, col, w, w, w,
                  pl.BlockSpec((1, D), lambda i: (0, 0))],
        out_specs=mat,
        out_shape=jax.ShapeDtypeStruct((N, D), jnp.float32),
    )(x, xne, z1, z2, dinv, dinv2, rdeg, wf, w1, w2, bf)


def kernel(X, edge_index, edge_weight, X_node_emb, W_gcn, W_ih_f, W_hh_f,
           b_ih_f, b_hh_f, W_ih_b, W_hh_b, b_ih_b, b_hh_b, W_com, W_fuse,
           b_fuse):
    pad = EPAD - E
    sidx = jnp.concatenate([edge_index[0], jnp.zeros((pad,), jnp.int32)])
    tidx = jnp.concatenate([edge_index[1], jnp.zeros((pad,), jnp.int32)])
    ewp = jnp.concatenate([edge_weight, jnp.zeros((pad,), jnp.float32)])

    degp = _deg_call(tidx.reshape(32, NBLK_A, BA), ewp.reshape(32, NBLK_A, BA))

    dinv, dinv2, rdeg, w1, w2 = _mid_call(
        degp, W_gcn, W_ih_f, b_ih_f, b_hh_f, W_ih_b, b_ih_b, b_hh_b,
        W_com, W_fuse)

    c1 = _coef_call(sidx.reshape(32, CHUNK_A), ewp.reshape(32, CHUNK_A),
                    dinv).reshape(EPAD)

    coefs = jnp.stack([c1, ewp]).reshape(2, 16, CHUNK_C)

    z = _spmm_call(sidx.reshape(16, CHUNK_C), tidx.reshape(16, NBLK_C, B),
                   coefs,
                   lax.bitcast_convert_type(
                       X.astype(jnp.bfloat16).reshape(N, D // 2, 2),
                       jnp.int32))

    fp = np.array(_FPERM, np.int32)
    return _fuse_call(X[:, fp], X_node_emb, z[0], z[1],
                      dinv.reshape(NPAD, 1), dinv2.reshape(NPAD, 1),
                      rdeg.reshape(NPAD, 1), W_fuse, w1[fp, :], w2[fp, :],
                      b_fuse.reshape(1, D))
